# baseline (speedup 1.0000x reference)
import functools

import jax
import jax.numpy as jnp
from jax import lax
from jax.experimental import pallas as pl
from jax.experimental.pallas import tpu as pltpu
from jax.experimental.pallas import tpu_sc as plsc


def _make_sc_gather(B, S, D, N, M):
    info = plsc.get_sparse_core_info()
    NC, NS, L = info.num_cores, info.num_subcores, info.num_lanes
    NW = NC * NS
    RPW = N // NW
    GC = 8

    mesh = plsc.VectorSubcoreMesh(core_axis_name="c", subcore_axis_name="s")

    @functools.partial(
        pl.kernel,
        out_type=jax.ShapeDtypeStruct((B * N, D), jnp.float32),
        mesh=mesh,
        scratch_types=[
            pltpu.VMEM((L,), jnp.int32),
            pltpu.VMEM((RPW,), jnp.int32),
            pltpu.VMEM((RPW,), jnp.int32),
            pltpu.VMEM((RPW,), jnp.int32),
            pltpu.VMEM((RPW, D), jnp.float32),
            pltpu.VMEM((RPW, D), jnp.float32),
            pltpu.VMEM((RPW,), jnp.int32),
            pltpu.VMEM((RPW,), jnp.int32),
            pltpu.SemaphoreType.DMA,
            pltpu.SemaphoreType.DMA,
            pltpu.SemaphoreType.DMA,
            pltpu.SemaphoreType.DMA,
        ],
    )
    def sc_gather(in_hbm, wsc_hbm, wgid_hbm, wpos_hbm, mstrows_hbm, out_hbm,
                  sc_v, gbase_v, pbase_v, gidx_v,
                  rows_a, rows_b, pos_a, pos_b,
                  gsem_a, gsem_b, ssem_a, ssem_b):
        wid = lax.axis_index("s") * NC + lax.axis_index("c")
        base = wid * RPW
        pltpu.sync_copy(wsc_hbm.at[pl.ds(wid * L, L)], sc_v)
        pltpu.sync_copy(wgid_hbm.at[pl.ds(base, RPW)], gbase_v)
        pltpu.sync_copy(wpos_hbm.at[pl.ds(base, RPW)], pbase_v)

        kv = sc_v[pl.ds(0, L)]
        k_w = kv[0]
        kpad = kv[1]

        pltpu.sync_copy(mstrows_hbm, rows_a)
        pltpu.sync_copy(mstrows_hbm, rows_b)

        bufs = [
            (rows_a, pos_a, gsem_a, ssem_a),
            (rows_b, pos_b, gsem_b, ssem_b),
        ]

        def two_batches(t, _):
            for p, (rows, pos, gsem, ssem) in enumerate(bufs):
                b = 2 * t + p

                @pl.when(b >= 2)
                def _():
                    pltpu.make_async_copy(
                        in_hbm.at[pl.ds(0, RPW)], rows, ssem).wait()

                for c in range(RPW // L):
                    sl = pl.ds(c * L, L)
                    gidx_v[sl] = gbase_v[sl] + b * S

                for i in range(RPW // GC):
                    @pl.when(i * GC < k_w)
                    def _(i=i):
                        pltpu.async_copy(
                            in_hbm.at[gidx_v.at[pl.ds(i * GC, GC)]],
                            rows.at[pl.ds(i * GC, GC)], gsem)
                for i in range(RPW // GC):
                    @pl.when(i * GC < k_w)
                    def _(i=i):
                        pltpu.make_async_copy(
                            in_hbm.at[pl.ds(0, GC)],
                            rows.at[pl.ds(i * GC, GC)], gsem).wait()

                for j in range(GC - 1):
                    @pl.when(k_w + j < kpad)
                    def _(j=j):
                        pltpu.async_copy(mstrows_hbm.at[pl.ds(0, 1)],
                                         rows.at[pl.ds(k_w + j, 1)], gsem)
                for j in range(GC - 1):
                    @pl.when(k_w + j < kpad)
                    def _(j=j):
                        pltpu.make_async_copy(
                            mstrows_hbm.at[pl.ds(0, 1)],
                            rows.at[pl.ds(0, 1)], gsem).wait()

                for c in range(RPW // L):
                    sl = pl.ds(c * L, L)
                    pos[sl] = pbase_v[sl] + b * N
                pltpu.async_copy(rows, out_hbm.at[pos], ssem)
            return 0

        lax.fori_loop(0, B // 2, two_batches, 0)
        pltpu.make_async_copy(in_hbm.at[pl.ds(0, RPW)], rows_a, ssem_a).wait()
        pltpu.make_async_copy(in_hbm.at[pl.ds(0, RPW)], rows_b, ssem_b).wait()

    return sc_gather


def kernel(inputs, mask_indices, un_masked_indices, mst):
    B, S, D = inputs.shape
    M = mask_indices.shape[0]
    N = M + un_masked_indices.shape[0]
    idx = jnp.concatenate([mask_indices, un_masked_indices]).astype(jnp.int32)

    info = plsc.get_sparse_core_info()
    NW = info.num_cores * info.num_subcores
    RPW = N // NW

    is_mask = (idx < M).astype(jnp.int32)
    order = jnp.argsort(is_mask, stable=True)
    sidx = idx[order]
    widx = sidx.reshape(RPW, NW).T
    wpos = order.astype(jnp.int32).reshape(RPW, NW).T.reshape(-1)
    spread = (jnp.arange(N, dtype=jnp.int32).reshape(NW, RPW) * 37) % S
    wgid = jnp.where(widx >= M, widx - M, spread).reshape(-1)
    k_w = jnp.sum((widx >= M).astype(jnp.int32), axis=1)
    kpad = jnp.minimum(((k_w + 7) // 8) * 8, RPW)
    wsc = jnp.zeros((NW, 16), jnp.int32)
    wsc = wsc.at[:, 0].set(k_w).at[:, 1].set(kpad).reshape(-1)
    mstrows = jnp.broadcast_to(mst.reshape(1, D), (RPW, D)).astype(inputs.dtype)

    sc_gather = _make_sc_gather(B, S, D, N, M)
    out_flat = sc_gather(inputs.reshape(B * S, D), wsc, wgid, wpos, mstrows)
    return out_flat.reshape(B, N, D)


# speedup vs baseline: 1.7482x; 1.7482x over previous
import functools

import jax
import jax.numpy as jnp
from jax import lax
from jax.experimental import pallas as pl
from jax.experimental.pallas import tpu as pltpu
from jax.experimental.pallas import tpu_sc as plsc


def _make_sc_gather(B, S, D, N, M):
    info = plsc.get_sparse_core_info()
    NC, NS, L = info.num_cores, info.num_subcores, info.num_lanes
    NW = NC * NS
    RPW = N // NW

    mesh = plsc.VectorSubcoreMesh(core_axis_name="c", subcore_axis_name="s")

    @functools.partial(
        pl.kernel,
        out_type=jax.ShapeDtypeStruct((B * N, D), jnp.float32),
        mesh=mesh,
        scratch_types=[
            pltpu.VMEM((RPW,), jnp.int32),
            pltpu.VMEM((RPW,), jnp.int32),
            pltpu.VMEM((RPW,), jnp.int32),
            pltpu.VMEM((RPW, D), jnp.float32),
            pltpu.VMEM((RPW, D), jnp.float32),
            pltpu.VMEM((D,), jnp.float32),
            pltpu.SemaphoreType.DMA,
            pltpu.SemaphoreType.DMA,
            pltpu.SemaphoreType.DMA,
        ],
    )
    def sc_gather(in_hbm, idx_hbm, gid_hbm, mst_hbm, out_hbm,
                  idx_v, gbase_v, gidx_v, rows_a, rows_b, mst_v,
                  gsem, ssem_a, ssem_b):
        wid = lax.axis_index("s") * NC + lax.axis_index("c")
        base = wid * RPW
        pltpu.sync_copy(idx_hbm.at[pl.ds(base, RPW)], idx_v)
        pltpu.sync_copy(gid_hbm.at[pl.ds(base, RPW)], gbase_v)
        pltpu.sync_copy(mst_hbm, mst_v)

        def do_batch(b, rows, ssem):
            ivecs = []
            for c in range(RPW // L):
                sl = pl.ds(c * L, L)
                ivecs.append(idx_v[sl])
                gidx_v[sl] = gbase_v[sl] + b * S
            pltpu.async_copy(in_hbm.at[gidx_v], rows, gsem).wait()

            for c in range(RPW // L):
                for l in range(L):
                    @pl.when(ivecs[c][l] < M)
                    def _():
                        j = c * L + l
                        for k in range(D // L):
                            rows[j, pl.ds(k * L, L)] = mst_v[pl.ds(k * L, L)]

            return pltpu.async_copy(
                rows, out_hbm.at[pl.ds(b * N + base, RPW)], ssem)

        def two_batches(t, _):
            cp_a = do_batch(2 * t, rows_a, ssem_a)
            cp_b = do_batch(2 * t + 1, rows_b, ssem_b)
            cp_a.wait()
            cp_b.wait()
            return 0

        lax.fori_loop(0, B // 2, two_batches, 0)

    return sc_gather


def kernel(inputs, mask_indices, un_masked_indices, mst):
    B, S, D = inputs.shape
    M = mask_indices.shape[0]
    N = M + un_masked_indices.shape[0]
    idx = jnp.concatenate([mask_indices, un_masked_indices]).astype(jnp.int32)
    spread = (jnp.arange(N, dtype=jnp.int32) * 37) % S
    gid = jnp.where(idx >= M, idx - M, spread)
    sc_gather = _make_sc_gather(B, S, D, N, M)
    out_flat = sc_gather(inputs.reshape(B * S, D), idx, gid,
                         mst.reshape(D).astype(inputs.dtype))
    return out_flat.reshape(B, N, D)
